# baseline (speedup 1.0000x reference)
import functools

import jax
import jax.numpy as jnp
from jax import lax
from jax.experimental import pallas as pl
from jax.experimental.pallas import tpu as pltpu
from jax.experimental.pallas import tpu_sc as plsc

_NC = 2
_NS = 16
_NW = _NC * _NS
_LANES = 16


_NBUF = 10


def _aggregate_sc(idx_flat, x, N, K, D, C, npw):
    mesh = plsc.VectorSubcoreMesh(core_axis_name="c", subcore_axis_name="s")
    G = C * K

    @functools.partial(
        pl.kernel,
        out_type=jax.ShapeDtypeStruct((N, D), jnp.float32),
        mesh=mesh,
        scratch_types=[
            pltpu.VMEM((npw * K,), jnp.int32),
            pltpu.VMEM((_NBUF, G, D), jnp.float32),
            pltpu.VMEM((_NBUF, C, D), jnp.float32),
        ] + [pltpu.SemaphoreType.DMA] * (2 * _NBUF),
    )
    def aggr_kernel(idx_hbm, x_hbm, out_hbm, idx_v, rows_v, out_v, *sems):
        wid = lax.axis_index("s") * _NC + lax.axis_index("c")
        gsem = sems[:_NBUF]
        osem = sems[_NBUF:]

        node_start = wid * npw
        cnt = jnp.minimum(N - node_start, npw)
        T = cnt // C
        copy_start = jnp.minimum(node_start, N - npw) * K
        off = node_start * K - copy_start

        def gather_desc(t, buf):
            return pltpu.make_async_copy(
                x_hbm.at[idx_v.at[pl.ds(off + t * G, G)]], rows_v.at[buf],
                gsem[buf])

        def out_desc(t, buf):
            return pltpu.make_async_copy(
                out_v.at[buf], out_hbm.at[pl.ds(node_start + t * C, C)],
                osem[buf])

        pltpu.sync_copy(idx_hbm.at[pl.ds(copy_start, npw * K)], idx_v)
        for buf in range(_NBUF):
            gather_desc(buf, buf).start()

        @pl.loop(0, T // _NBUF)
        def _(i):
            for buf in range(_NBUF):
                t = i * _NBUF + buf
                gather_desc(t, buf).wait()

                @pl.when(i > 0)
                def _():
                    out_desc(t, buf).wait()

                for n in range(C):
                    base = n * K
                    accs = tuple(
                        rows_v[buf, base, pl.ds(d * _LANES, _LANES)]
                        for d in range(D // _LANES)
                    )

                    def body(k, a, base=base):
                        return tuple(
                            v + rows_v[buf, base + k, pl.ds(d * _LANES, _LANES)]
                            for d, v in enumerate(a)
                        )

                    accs = lax.fori_loop(1, K, body, accs)
                    for d, v in enumerate(accs):
                        out_v[buf, n, pl.ds(d * _LANES, _LANES)] = v

                out_desc(t, buf).start()

                @pl.when(t + _NBUF < T)
                def _():
                    gather_desc(t + _NBUF, buf).start()

        for buf in range(_NBUF):
            out_desc(T - _NBUF + buf, buf).wait()

    return aggr_kernel(idx_flat, x)


def _matmul_bias_tc(x, wt, bias, N, D):
    BLK = 1000

    def body(x_ref, w_ref, b_ref, o_ref):
        o_ref[...] = jnp.dot(
            x_ref[...], w_ref[...],
            preferred_element_type=jnp.float32) + b_ref[...]

    return pl.pallas_call(
        body,
        grid=(N // BLK,),
        in_specs=[
            pl.BlockSpec((BLK, D), lambda i: (i, 0)),
            pl.BlockSpec((D, D), lambda i: (0, 0)),
            pl.BlockSpec((1, D), lambda i: (0, 0)),
        ],
        out_specs=pl.BlockSpec((BLK, D), lambda i: (i, 0)),
        out_shape=jax.ShapeDtypeStruct((N, D), jnp.float32),
    )(x, wt, bias)


def _combine_tc(out1, aggr, w2t, N, D):
    BLK = 1000

    def body(o1_ref, a_ref, w_ref, o_ref):
        acc = jnp.dot(a_ref[...], w_ref[...],
                      preferred_element_type=jnp.float32)
        o_ref[...] = jnp.maximum(acc + o1_ref[...], 0.0)

    return pl.pallas_call(
        body,
        grid=(N // BLK,),
        in_specs=[
            pl.BlockSpec((BLK, D), lambda i: (i, 0)),
            pl.BlockSpec((BLK, D), lambda i: (i, 0)),
            pl.BlockSpec((D, D), lambda i: (0, 0)),
        ],
        out_specs=pl.BlockSpec((BLK, D), lambda i: (i, 0)),
        out_shape=jax.ShapeDtypeStruct((N, D), jnp.float32),
    )(out1, aggr, w2t)


def kernel(neighbours, shape_features, W1, b1, W2, b2):
    N, K = neighbours.shape
    D = shape_features.shape[1]

    C = 64 // K
    npw = -(-N // _NW)
    npw = -(-npw // (_NBUF * C)) * (_NBUF * C)

    idx_flat = neighbours.reshape(-1).astype(jnp.int32)
    aggr = _aggregate_sc(idx_flat, shape_features, N, K, D, C, npw)

    bias = (b1 + b2).reshape(1, D)
    out1 = _matmul_bias_tc(shape_features, W1.T, bias, N, D)
    return _combine_tc(out1, aggr, W2.T, N, D)


# speedup vs baseline: 1.0526x; 1.0526x over previous
import functools

import jax
import jax.numpy as jnp
from jax import lax
from jax.experimental import pallas as pl
from jax.experimental.pallas import tpu as pltpu
from jax.experimental.pallas import tpu_sc as plsc

_NC = 2
_NS = 16
_NW = _NC * _NS
_LANES = 16


_NBUF = 2


def _aggregate_sc(idx_flat, x, N, K, D, C, npw):
    mesh = plsc.VectorSubcoreMesh(core_axis_name="c", subcore_axis_name="s")
    G = C * K

    @functools.partial(
        pl.kernel,
        out_type=jax.ShapeDtypeStruct((N, D), jnp.float32),
        mesh=mesh,
        scratch_types=[
            pltpu.VMEM((npw * K,), jnp.int32),
            pltpu.VMEM((_NBUF, G, D), jnp.float32),
            pltpu.VMEM((_NBUF, C, D), jnp.float32),
            pltpu.VMEM_SHARED((N, D), jnp.float32),
        ] + [pltpu.SemaphoreType.DMA] * (2 * _NBUF),
    )
    def aggr_kernel(idx_hbm, x_hbm, out_hbm, idx_v, rows_v, out_v, x_sh,
                    *sems):
        wid = lax.axis_index("s") * _NC + lax.axis_index("c")
        gsem = sems[:_NBUF]
        osem = sems[_NBUF:]

        ch = -(-(-(-N // _NS)) // 8) * 8
        cstart = jnp.minimum(lax.axis_index("s") * ch, N - ch)
        pltpu.sync_copy(x_hbm.at[pl.ds(cstart, ch)],
                        x_sh.at[pl.ds(cstart, ch)])
        plsc.subcore_barrier()

        node_start = wid * npw
        cnt = jnp.minimum(N - node_start, npw)
        T = cnt // C
        copy_start = jnp.minimum(node_start, N - npw) * K
        off = node_start * K - copy_start

        def gather_desc(t, buf):
            return pltpu.make_async_copy(
                x_sh.at[idx_v.at[pl.ds(off + t * G, G)]], rows_v.at[buf],
                gsem[buf])

        def out_desc(t, buf):
            return pltpu.make_async_copy(
                out_v.at[buf], out_hbm.at[pl.ds(node_start + t * C, C)],
                osem[buf])

        pltpu.sync_copy(idx_hbm.at[pl.ds(copy_start, npw * K)], idx_v)
        for buf in range(_NBUF):
            gather_desc(buf, buf).start()

        @pl.loop(0, T // _NBUF)
        def _(i):
            for buf in range(_NBUF):
                t = i * _NBUF + buf
                gather_desc(t, buf).wait()

                @pl.when(i > 0)
                def _():
                    out_desc(t, buf).wait()

                for n in range(C):
                    base = n * K
                    accs = tuple(
                        rows_v[buf, base, pl.ds(d * _LANES, _LANES)]
                        for d in range(D // _LANES)
                    )

                    def body(k, a, base=base):
                        return tuple(
                            v + rows_v[buf, base + k, pl.ds(d * _LANES, _LANES)]
                            for d, v in enumerate(a)
                        )

                    accs = lax.fori_loop(1, K, body, accs)
                    for d, v in enumerate(accs):
                        out_v[buf, n, pl.ds(d * _LANES, _LANES)] = v

                out_desc(t, buf).start()

                @pl.when(t + _NBUF < T)
                def _():
                    gather_desc(t + _NBUF, buf).start()

        for buf in range(_NBUF):
            out_desc(T - _NBUF + buf, buf).wait()

    return aggr_kernel(idx_flat, x)


def _matmul_bias_tc(x, wt, bias, N, D):
    BLK = 1000

    def body(x_ref, w_ref, b_ref, o_ref):
        o_ref[...] = jnp.dot(
            x_ref[...], w_ref[...],
            preferred_element_type=jnp.float32) + b_ref[...]

    return pl.pallas_call(
        body,
        grid=(N // BLK,),
        in_specs=[
            pl.BlockSpec((BLK, D), lambda i: (i, 0)),
            pl.BlockSpec((D, D), lambda i: (0, 0)),
            pl.BlockSpec((1, D), lambda i: (0, 0)),
        ],
        out_specs=pl.BlockSpec((BLK, D), lambda i: (i, 0)),
        out_shape=jax.ShapeDtypeStruct((N, D), jnp.float32),
    )(x, wt, bias)


def _combine_tc(out1, aggr, w2t, N, D):
    BLK = 1000

    def body(o1_ref, a_ref, w_ref, o_ref):
        acc = jnp.dot(a_ref[...], w_ref[...],
                      preferred_element_type=jnp.float32)
        o_ref[...] = jnp.maximum(acc + o1_ref[...], 0.0)

    return pl.pallas_call(
        body,
        grid=(N // BLK,),
        in_specs=[
            pl.BlockSpec((BLK, D), lambda i: (i, 0)),
            pl.BlockSpec((BLK, D), lambda i: (i, 0)),
            pl.BlockSpec((D, D), lambda i: (0, 0)),
        ],
        out_specs=pl.BlockSpec((BLK, D), lambda i: (i, 0)),
        out_shape=jax.ShapeDtypeStruct((N, D), jnp.float32),
    )(out1, aggr, w2t)


def kernel(neighbours, shape_features, W1, b1, W2, b2):
    N, K = neighbours.shape
    D = shape_features.shape[1]

    C = 128 // K
    npw = -(-N // _NW)
    npw = -(-npw // (_NBUF * C)) * (_NBUF * C)

    idx_flat = neighbours.reshape(-1).astype(jnp.int32)
    aggr = _aggregate_sc(idx_flat, shape_features, N, K, D, C, npw)

    bias = (b1 + b2).reshape(1, D)
    out1 = _matmul_bias_tc(shape_features, W1.T, bias, N, D)
    return _combine_tc(out1, aggr, W2.T, N, D)
